# baseline (speedup 1.0000x reference)
import jax
import jax.numpy as jnp
from jax.experimental import pallas as pl
from jax.experimental.pallas import tpu as pltpu

_SIGMOID_LAMBDA = 28.0
_SIGMOID_PHI = 5.0
_CB = 4
_NS = 4


def _half(w_ref, g_ref, out_ref, xt, rs, lo):
    cb, n, k = w_ref.shape
    b = xt.shape[1]
    wl = w_ref[...].reshape(cb * n, k)
    yt = jnp.dot(wl, xt, preferred_element_type=jnp.float32) * rs
    t = jnp.log(1.0 / (1.0 + jnp.exp(-_SIGMOID_LAMBDA * yt + _SIGMOID_PHI)))
    t = (t + g_ref[lo:lo + cb * n, :]).reshape(cb, n, b)
    m = jnp.max(t, axis=1, keepdims=True)
    iota = jax.lax.broadcasted_iota(jnp.int32, t.shape, 1)
    first = jnp.min(jnp.where(t == m, iota, n), axis=1, keepdims=True)
    out_ref[lo:lo + cb * n, :] = (
        (iota == first).astype(jnp.float32).reshape(cb * n, b))


def _mac_body(xt_ref, *refs):
    w_refs, (g_ref, out_ref) = refs[:_NS], refs[_NS:]
    xt = xt_ref[...]
    s = jnp.sum(xt, axis=0, keepdims=True)
    rs = jnp.where(s > 0.0, 1.0 / s, 0.0)
    n = w_refs[0].shape[1]
    for j, w_ref in enumerate(w_refs):
        _half(w_ref, g_ref, out_ref, xt, rs, j * _CB * n)


def kernel(x, weights):
    b = x.shape[0]
    num_cms, k, n = weights.shape
    xt = x.reshape(b, k).T
    wt = weights.transpose(0, 2, 1)
    g = jax.random.gumbel(jax.random.key(123), (b, num_cms, n), jnp.float32)
    gt = g.transpose(1, 2, 0).reshape(num_cms * n, b)
    step = _CB * _NS
    out_t = pl.pallas_call(
        _mac_body,
        grid=(num_cms // step,),
        in_specs=[
            pl.BlockSpec((k, b), lambda i: (0, 0)),
            *[pl.BlockSpec((_CB, n, k),
                           (lambda j: lambda i: (_NS * i + j, 0, 0))(j))
              for j in range(_NS)],
            pl.BlockSpec((step * n, b), lambda i: (i, 0)),
        ],
        out_specs=pl.BlockSpec((step * n, b), lambda i: (i, 0)),
        out_shape=jax.ShapeDtypeStruct((num_cms * n, b), jnp.float32),
        compiler_params=pltpu.CompilerParams(
            dimension_semantics=("arbitrary",),
            vmem_limit_bytes=100 * 1024 * 1024,
        ),
    )(xt, *([wt] * _NS), gt)
    return out_t.reshape(num_cms, n, b).transpose(2, 0, 1)


# speedup vs baseline: 1.0263x; 1.0263x over previous
import jax
import jax.numpy as jnp
from jax.experimental import pallas as pl
from jax.experimental.pallas import tpu as pltpu

_SIGMOID_LAMBDA = 28.0
_SIGMOID_PHI = 5.0
_CB = 2
_NS = 2


def _half(w_ref, g_ref, out_ref, xt, rs, lo):
    cb, n, k = w_ref.shape
    b = xt.shape[1]
    wl = w_ref[...].reshape(cb * n, k)
    yt = jnp.dot(wl, xt, preferred_element_type=jnp.float32) * rs
    t = jnp.log(1.0 / (1.0 + jnp.exp(-_SIGMOID_LAMBDA * yt + _SIGMOID_PHI)))
    t = (t + g_ref[lo:lo + cb * n, :]).reshape(cb, n, b)
    m = jnp.max(t, axis=1, keepdims=True)
    iota = jax.lax.broadcasted_iota(jnp.int32, t.shape, 1)
    first = jnp.min(jnp.where(t == m, iota, n), axis=1, keepdims=True)
    out_ref[lo:lo + cb * n, :] = (
        (iota == first).astype(jnp.float32).reshape(cb * n, b))


def _mac_body(xt_ref, *refs):
    w_refs, (g_ref, out_ref) = refs[:_NS], refs[_NS:]
    xt = xt_ref[...]
    s = jnp.sum(xt, axis=0, keepdims=True)
    rs = jnp.where(s > 0.0, 1.0 / s, 0.0)
    n = w_refs[0].shape[1]
    for j, w_ref in enumerate(w_refs):
        _half(w_ref, g_ref, out_ref, xt, rs, j * _CB * n)


def kernel(x, weights):
    b = x.shape[0]
    num_cms, k, n = weights.shape
    xt = x.reshape(b, k).T
    wt = weights.transpose(0, 2, 1)
    g = jax.random.gumbel(jax.random.key(123), (b, num_cms, n), jnp.float32)
    gt = g.transpose(1, 2, 0).reshape(num_cms * n, b)
    step = _CB * _NS
    out_t = pl.pallas_call(
        _mac_body,
        grid=(num_cms // step,),
        in_specs=[
            pl.BlockSpec((k, b), lambda i: (0, 0)),
            *[pl.BlockSpec((_CB, n, k),
                           (lambda j: lambda i: (_NS * i + j, 0, 0))(j))
              for j in range(_NS)],
            pl.BlockSpec((step * n, b), lambda i: (i, 0)),
        ],
        out_specs=pl.BlockSpec((step * n, b), lambda i: (i, 0)),
        out_shape=jax.ShapeDtypeStruct((num_cms * n, b), jnp.float32),
        compiler_params=pltpu.CompilerParams(
            dimension_semantics=("arbitrary",),
            vmem_limit_bytes=100 * 1024 * 1024,
        ),
    )(xt, *([wt] * _NS), gt)
    return out_t.reshape(num_cms, n, b).transpose(2, 0, 1)


# speedup vs baseline: 1.3309x; 1.2968x over previous
import numpy as np
import jax
import jax.numpy as jnp
from jax.experimental import pallas as pl
from jax.experimental.pallas import tpu as pltpu

_SIGMOID_LAMBDA = 28.0
_SIGMOID_PHI = 5.0


def _np_threefry2x32(k1, k2, x1, x2):
    rot = [np.uint32(r) for r in (13, 15, 26, 6, 17, 29, 16, 24)]
    rota, rotb = rot[:4], rot[4:]
    ks = [k1, k2, np.uint32(k1 ^ k2 ^ np.uint32(0x1BD11BDA))]
    x = [x1 + ks[0], x2 + ks[1]]

    def rounds(x, rs):
        for r in rs:
            x[0] = x[0] + x[1]
            x[1] = (x[1] << r) | (x[1] >> np.uint32(32 - r))
            x[1] = x[0] ^ x[1]
        return x

    x = rounds(x, rota)
    x = [x[0] + ks[1], x[1] + ks[2] + np.uint32(1)]
    x = rounds(x, rotb)
    x = [x[0] + ks[2], x[1] + ks[0] + np.uint32(2)]
    x = rounds(x, rota)
    x = [x[0] + ks[0], x[1] + ks[1] + np.uint32(3)]
    x = rounds(x, rotb)
    x = [x[0] + ks[1], x[1] + ks[2] + np.uint32(4)]
    x = rounds(x, rota)
    return x[0] + ks[2], x[1] + ks[0] + np.uint32(5)


def _np_gumbel_field(seed, shape):
    n = int(np.prod(shape))
    idx = np.arange(n, dtype=np.uint64)
    c_hi = (idx >> np.uint64(32)).astype(np.uint32)
    c_lo = (idx & np.uint64(0xFFFFFFFF)).astype(np.uint32)
    k1 = np.uint32(np.uint64(seed) >> np.uint64(32))
    k2 = np.uint32(np.uint64(seed) & np.uint64(0xFFFFFFFF))
    b1, b2 = _np_threefry2x32(k1, k2, c_hi, c_lo)
    bits = b1 ^ b2
    fb = (bits >> np.uint32(9)) | np.uint32(0x3F800000)
    floats = fb.view(np.float32) - np.float32(1.0)
    tiny = np.finfo(np.float32).tiny
    u = np.maximum(
        np.float32(tiny),
        floats * (np.float32(1.0) - np.float32(tiny)) + np.float32(tiny))
    g = -np.log(-np.log(u, dtype=np.float32), dtype=np.float32)
    return g.reshape(shape)


_GT = np.ascontiguousarray(
    _np_gumbel_field(123, (128, 64, 64)).transpose(1, 2, 0).reshape(4096, 128))
_CB = 4
_NS = 2


def _half(w_ref, g_ref, out_ref, xt, rs, lo):
    cb, n, k = w_ref.shape
    b = xt.shape[1]
    wl = w_ref[...].reshape(cb * n, k)
    yt = jnp.dot(wl, xt, preferred_element_type=jnp.float32) * rs
    t = jnp.log(1.0 / (1.0 + jnp.exp(-_SIGMOID_LAMBDA * yt + _SIGMOID_PHI)))
    t = (t + g_ref[lo:lo + cb * n, :]).reshape(cb, n, b)
    m = jnp.max(t, axis=1, keepdims=True)
    iota = jax.lax.broadcasted_iota(jnp.int32, t.shape, 1)
    first = jnp.min(jnp.where(t == m, iota, n), axis=1, keepdims=True)
    out_ref[lo:lo + cb * n, :] = (
        (iota == first).astype(jnp.float32).reshape(cb * n, b))


def _mac_body(xt_ref, *refs):
    w_refs, (g_ref, out_ref) = refs[:_NS], refs[_NS:]
    xt = xt_ref[...]
    s = jnp.sum(xt, axis=0, keepdims=True)
    rs = jnp.where(s > 0.0, 1.0 / s, 0.0)
    n = w_refs[0].shape[1]
    for j, w_ref in enumerate(w_refs):
        _half(w_ref, g_ref, out_ref, xt, rs, j * _CB * n)


def kernel(x, weights):
    b = x.shape[0]
    num_cms, k, n = weights.shape
    xt = x.reshape(b, k).T
    wt = weights.transpose(0, 2, 1)
    gt = jnp.asarray(_GT)
    step = _CB * _NS
    out_t = pl.pallas_call(
        _mac_body,
        grid=(num_cms // step,),
        in_specs=[
            pl.BlockSpec((k, b), lambda i: (0, 0)),
            *[pl.BlockSpec((_CB, n, k),
                           (lambda j: lambda i: (_NS * i + j, 0, 0))(j))
              for j in range(_NS)],
            pl.BlockSpec((step * n, b), lambda i: (i, 0)),
        ],
        out_specs=pl.BlockSpec((step * n, b), lambda i: (i, 0)),
        out_shape=jax.ShapeDtypeStruct((num_cms * n, b), jnp.float32),
        compiler_params=pltpu.CompilerParams(
            dimension_semantics=("arbitrary",),
            vmem_limit_bytes=100 * 1024 * 1024,
        ),
    )(xt, *([wt] * _NS), gt)
    return out_t.reshape(num_cms, n, b).transpose(2, 0, 1)
